# baseline (device time: 43991 ns/iter reference)
import jax
import jax.numpy as jnp
from jax import lax
from jax.experimental import pallas as pl
from jax.experimental.pallas import tpu as pltpu

N_DEV = 4
N_STREAMS = 4


def _gelu(y):
    c = 0.7978845608028654
    return 0.5 * y * (1.0 + jnp.tanh(c * (y + 0.044715 * y * y * y)))


def kernel(x, w_mat):
    m, k_per = x.shape
    _, n = w_mat.shape
    chunk = m // N_DEV
    scol = n // N_STREAMS

    def body(x_ref, w_ref, out_ref, comm_ref, send_sems, recv_sems):
        my = lax.axis_index("i")
        left = lax.rem(my + N_DEV - 1, N_DEV)
        right = lax.rem(my + 1, N_DEV)

        barrier = pltpu.get_barrier_semaphore()
        for nbr in (left, right):
            pl.semaphore_signal(
                barrier, inc=1,
                device_id=(nbr,), device_id_type=pl.DeviceIdType.MESH,
            )
        pl.semaphore_wait(barrier, 2)

        def gemm_chunk(c, cols=None):
            rows = pl.ds(c * chunk, chunk)
            if cols is None:
                out_ref[rows, :] = jnp.dot(
                    x_ref[rows, :], w_ref[:, :],
                    preferred_element_type=jnp.float32,
                )
            else:
                out_ref[rows, cols] = jnp.dot(
                    x_ref[rows, :], w_ref[:, cols],
                    preferred_element_type=jnp.float32,
                )

        def cols_of(k):
            return pl.ds(k * scol, scol)

        def is_cw(k):
            return k < N_STREAMS // 2

        def make_rs(k, s):
            if is_cw(k):
                c_send = lax.rem(my + N_DEV - s, N_DEV)
                tgt = right
            else:
                c_send = lax.rem(my + s, N_DEV)
                tgt = left
            return pltpu.make_async_remote_copy(
                src_ref=out_ref.at[pl.ds(c_send * chunk, chunk), cols_of(k)],
                dst_ref=comm_ref.at[k, s],
                send_sem=send_sems.at[k, s],
                recv_sem=recv_sems.at[k, s],
                device_id=(tgt,),
                device_id_type=pl.DeviceIdType.MESH,
            )

        def rs_recv_chunk(k, s):
            if is_cw(k):
                return lax.rem(my + N_DEV - s - 1, N_DEV)
            return lax.rem(my + s + 1, N_DEV)

        def own_chunk(k):
            return lax.rem(my + 1, N_DEV) if is_cw(k) else left

        hc = chunk // 2

        def make_ag(k, slot):
            own = own_chunk(k)
            if slot == 0:
                rows, tgt = pl.ds(own * chunk, chunk), right
            elif slot == 1:
                rows, tgt = pl.ds(own * chunk, chunk), left
            elif slot == 2:
                c = lax.rem(own + N_DEV - 1, N_DEV)
                rows, tgt = pl.ds(c * chunk, hc), right
            else:
                c = lax.rem(own + 1, N_DEV)
                rows, tgt = pl.ds(c * chunk + hc, hc), left
            return pltpu.make_async_remote_copy(
                src_ref=out_ref.at[rows, cols_of(k)],
                dst_ref=out_ref.at[rows, cols_of(k)],
                send_sem=send_sems.at[k, N_DEV - 1 + slot],
                recv_sem=recv_sems.at[k, N_DEV - 1 + slot],
                device_id=(tgt,),
                device_id_type=pl.DeviceIdType.MESH,
            )

        half_s = N_STREAMS // 2
        ORDER = tuple(
            k for pair in zip(range(half_s), range(half_s, N_STREAMS)) for k in pair
        )
        rs = {k: [make_rs(k, s) for s in range(N_DEV - 1)] for k in range(N_STREAMS)}
        ag = {k: [make_ag(k, slot) for slot in range(4)] for k in range(N_STREAMS)}

        def acc(k, s):
            rows = pl.ds(rs_recv_chunk(k, s) * chunk, chunk)
            c = cols_of(k)
            out_ref[rows, c] = out_ref[rows, c] + comm_ref[k, s]

        cw_half = pl.ds(0, n // 2)
        ccw_half = pl.ds(n // 2, n // 2)
        gemm_chunk(my, cw_half)
        for k in range(half_s):
            rs[k][0].start()
        gemm_chunk(my, ccw_half)
        for k in range(half_s, N_STREAMS):
            rs[k][0].start()
        gemm_chunk(left)
        gemm_chunk(right)
        gemm_chunk(lax.rem(my + 2, N_DEV))

        for s in range(N_DEV - 2):
            for k in ORDER:
                rs[k][s].wait_recv()
                acc(k, s)
                rs[k][s + 1].start()
        for k in ORDER:
            rs[k][N_DEV - 2].wait_recv()
            acc(k, N_DEV - 2)
            rows = pl.ds(own_chunk(k) * chunk, chunk)
            c = cols_of(k)
            out_ref[rows, c] = _gelu(out_ref[rows, c])
            ag[k][0].start()
            ag[k][1].start()

        for k in ORDER:
            ag[k][0].wait_recv()
            ag[k][1].wait_recv()
            ag[k][2].start()
            ag[k][3].start()
        for k in ORDER:
            ag[k][2].wait_recv()
            ag[k][3].wait_recv()

        for k in range(N_STREAMS):
            for r in rs[k] + ag[k]:
                r.wait_send()

    n_hops = (N_DEV - 1) + 4
    return pl.pallas_call(
        body,
        out_shape=jax.ShapeDtypeStruct((m, n), jnp.float32),
        in_specs=[
            pl.BlockSpec(memory_space=pltpu.VMEM),
            pl.BlockSpec(memory_space=pltpu.VMEM),
        ],
        out_specs=pl.BlockSpec(memory_space=pltpu.VMEM),
        scratch_shapes=[
            pltpu.VMEM((N_STREAMS, N_DEV - 1, chunk, scol), jnp.float32),
            pltpu.SemaphoreType.DMA((N_STREAMS, n_hops)),
            pltpu.SemaphoreType.DMA((N_STREAMS, n_hops)),
        ],
        compiler_params=pltpu.CompilerParams(collective_id=0),
    )(x, w_mat)


# device time: 43827 ns/iter; 1.0037x vs baseline; 1.0037x over previous
import jax
import jax.numpy as jnp
from jax import lax
from jax.experimental import pallas as pl
from jax.experimental.pallas import tpu as pltpu

N_DEV = 4
N_STREAMS = 4


def _gelu(y):
    c = 0.7978845608028654
    return 0.5 * y * (1.0 + jnp.tanh(c * (y + 0.044715 * y * y * y)))


def kernel(x, w_mat):
    m, k_per = x.shape
    _, n = w_mat.shape
    chunk = m // N_DEV
    scol = n // N_STREAMS
    hc = chunk // 2

    def body(x_ref, w_ref, out_ref, g1_ref, g2_ref, send_sems, recv_sems):
        my = lax.axis_index("i")
        left = lax.rem(my + N_DEV - 1, N_DEV)
        right = lax.rem(my + 1, N_DEV)
        my2 = lax.rem(my + 2, N_DEV)

        barrier = pltpu.get_barrier_semaphore()
        for nbr in (left, right):
            pl.semaphore_signal(
                barrier, inc=1,
                device_id=(nbr,), device_id_type=pl.DeviceIdType.MESH,
            )
        pl.semaphore_wait(barrier, 2)

        def gemm_chunk(c, cols=None):
            rows = pl.ds(c * chunk, chunk)
            if cols is None:
                out_ref[rows, :] = jnp.dot(
                    x_ref[rows, :], w_ref[:, :],
                    preferred_element_type=jnp.float32,
                )
            else:
                out_ref[rows, cols] = jnp.dot(
                    x_ref[rows, :], w_ref[:, cols],
                    preferred_element_type=jnp.float32,
                )

        def cols_of(k):
            return pl.ds(k * scol, scol)

        def is_cw(k):
            return k < N_STREAMS // 2

        def rdma(src, dst, k, slot, tgt):
            return pltpu.make_async_remote_copy(
                src_ref=src, dst_ref=dst,
                send_sem=send_sems.at[k, slot],
                recv_sem=recv_sems.at[k, slot],
                device_id=(tgt,),
                device_id_type=pl.DeviceIdType.MESH,
            )

        def make_rs1(k):
            c = cols_of(k)
            c1 = left if is_cw(k) else right
            h0 = out_ref.at[pl.ds(c1 * chunk, hc), c]
            h1 = out_ref.at[pl.ds(c1 * chunk + hc, hc), c]
            srcL, srcR = (h0, h1) if is_cw(k) else (h1, h0)
            return (
                rdma(srcL, g1_ref.at[k, 1], k, 0, left),
                rdma(srcR, g1_ref.at[k, 0], k, 1, right),
            )

        def make_rs2(k):
            c = cols_of(k)
            r_my = out_ref.at[pl.ds(my * chunk, chunk), c]
            r_my2 = out_ref.at[pl.ds(my2 * chunk, chunk), c]
            srcL, srcR = (r_my, r_my2) if is_cw(k) else (r_my2, r_my)
            return (
                rdma(srcL, g2_ref.at[k, 1], k, 2, left),
                rdma(srcR, g2_ref.at[k, 0], k, 3, right),
            )

        def own_chunk(k):
            return right if is_cw(k) else left

        def make_ag(k, slot):
            c = cols_of(k)
            own = own_chunk(k)
            if slot == 0:
                rows, tgt = pl.ds(own * chunk, chunk), right
            elif slot == 1:
                rows, tgt = pl.ds(own * chunk, chunk), left
            elif slot == 2:
                cc = lax.rem(own + N_DEV - 1, N_DEV)
                rows, tgt = pl.ds(cc * chunk, hc), right
            else:
                cc = lax.rem(own + 1, N_DEV)
                rows, tgt = pl.ds(cc * chunk + hc, hc), left
            return rdma(
                out_ref.at[rows, c], out_ref.at[rows, c], k, 4 + slot, tgt
            )

        ORDER = (0, 2, 1, 3)
        rs1 = {k: make_rs1(k) for k in range(N_STREAMS)}
        rs2 = {k: make_rs2(k) for k in range(N_STREAMS)}
        ag = {k: [make_ag(k, s) for s in range(4)] for k in range(N_STREAMS)}

        cw_half = pl.ds(0, n // 2)
        ccw_half = pl.ds(n // 2, n // 2)
        gemm_chunk(left, cw_half)
        for k in range(N_STREAMS // 2):
            rs1[k][0].start()
            rs1[k][1].start()
        gemm_chunk(right, ccw_half)
        for k in range(N_STREAMS // 2, N_STREAMS):
            rs1[k][0].start()
            rs1[k][1].start()
        gemm_chunk(my)
        gemm_chunk(my2)
        gemm_chunk(right, cw_half)
        gemm_chunk(left, ccw_half)

        rows_h0_my = pl.ds(my * chunk, hc)
        rows_h1_my2 = pl.ds(my2 * chunk + hc, hc)
        for k in ORDER:
            c = cols_of(k)
            if is_cw(k):
                rs1[k][0].wait_recv()
                out_ref[rows_h0_my, c] = out_ref[rows_h0_my, c] + g1_ref[k, 1]
                rs2[k][0].start()
                rs1[k][1].wait_recv()
                out_ref[rows_h1_my2, c] = out_ref[rows_h1_my2, c] + g1_ref[k, 0]
                rs2[k][1].start()
            else:
                rs1[k][1].wait_recv()
                out_ref[rows_h0_my, c] = out_ref[rows_h0_my, c] + g1_ref[k, 0]
                rs2[k][1].start()
                rs1[k][0].wait_recv()
                out_ref[rows_h1_my2, c] = out_ref[rows_h1_my2, c] + g1_ref[k, 1]
                rs2[k][0].start()

        for k in ORDER:
            c = cols_of(k)
            rows = pl.ds(own_chunk(k) * chunk, chunk)
            rs2[k][0].wait_recv()
            rs2[k][1].wait_recv()
            out_ref[rows, c] = _gelu(
                out_ref[rows, c] + g2_ref[k, 0] + g2_ref[k, 1]
            )
            ag[k][0].start()
            ag[k][1].start()

        for k in ORDER:
            ag[k][0].wait_recv()
            ag[k][1].wait_recv()
            ag[k][2].start()
            ag[k][3].start()
        for k in ORDER:
            ag[k][2].wait_recv()
            ag[k][3].wait_recv()

        for k in range(N_STREAMS):
            for r in (*rs1[k], *rs2[k], *ag[k]):
                r.wait_send()

    return pl.pallas_call(
        body,
        out_shape=jax.ShapeDtypeStruct((m, n), jnp.float32),
        in_specs=[
            pl.BlockSpec(memory_space=pltpu.VMEM),
            pl.BlockSpec(memory_space=pltpu.VMEM),
        ],
        out_specs=pl.BlockSpec(memory_space=pltpu.VMEM),
        scratch_shapes=[
            pltpu.VMEM((N_STREAMS, 2, hc, scol), jnp.float32),
            pltpu.VMEM((N_STREAMS, 2, chunk, scol), jnp.float32),
            pltpu.SemaphoreType.DMA((N_STREAMS, 8)),
            pltpu.SemaphoreType.DMA((N_STREAMS, 8)),
        ],
        compiler_params=pltpu.CompilerParams(collective_id=0),
    )(x, w_mat)
